# Initial kernel scaffold; baseline (speedup 1.0000x reference)
import jax
import jax.numpy as jnp
from jax.experimental import pallas as pl


def kernel(x, edge_index, W_self1, W_neigh1, b1, W_self2, W_neigh2, b2, W_pred, b_pred):
    raise NotImplementedError("write your pallas kernel here")


# speedup vs baseline: 7.8536x; 7.8536x over previous
import functools

import jax
import jax.numpy as jnp
from jax import lax
from jax.experimental import pallas as pl
from jax.experimental.pallas import tpu as pltpu
from jax.experimental.pallas import tpu_sc as plsc

_N = 10000
_E = 320000
_D = 128
_NCLS = 32
_NC = 2
_NS = 16
_NW = _NC * _NS
_EPT = _E // _NW
_CK = 80
_NCH = _EPT // _CK
_RPS = _N // _NS
_W1 = 136
_RB = 1000


def _make_sc_agg(width):
    mesh = plsc.VectorSubcoreMesh(core_axis_name="c", subcore_axis_name="s")

    @functools.partial(
        pl.kernel,
        out_type=jax.ShapeDtypeStruct((_NC, _N, width), jnp.float32),
        mesh=mesh,
        compiler_params=pltpu.CompilerParams(use_tc_tiling_on_sc=False),
        scratch_types=[
            pltpu.VMEM((_NCH, _CK), jnp.int32),
            pltpu.VMEM((_NCH, _CK), jnp.int32),
            pltpu.VMEM((_CK, width), jnp.float32),
            pltpu.VMEM((_CK, width), jnp.float32),
            pltpu.VMEM_SHARED((_N, width), jnp.float32),
            pltpu.SemaphoreType.DMA,
            pltpu.SemaphoreType.DMA,
        ],
    )
    def sc_agg(table, src, dst, zeros, out, src_v, dst_v, rows0, rows1, acc,
               sem0, sem1):
        cid = lax.axis_index("c")
        sid = lax.axis_index("s")
        wid = cid * _NS + sid

        pltpu.sync_copy(zeros.at[pl.ds(sid * _RPS, _RPS)],
                        acc.at[pl.ds(sid * _RPS, _RPS)])
        pltpu.sync_copy(src.at[wid], src_v)
        pltpu.sync_copy(dst.at[wid], dst_v)
        plsc.subcore_barrier()

        def gather(c, buf, sem):
            pltpu.async_copy(table.at[src_v.at[c]], buf, sem)

        def wait(c, buf, sem):
            pltpu.make_async_copy(table.at[src_v.at[c]], buf, sem).wait()

        def scat(c, buf):
            pltpu.sync_copy(buf, acc.at[dst_v.at[c]], add=True)

        gather(0, rows0, sem0)

        def body(i, carry):
            c0 = 2 * i
            gather(c0 + 1, rows1, sem1)
            wait(c0, rows0, sem0)
            scat(c0, rows0)
            gather(c0 + 2, rows0, sem0)
            wait(c0 + 1, rows1, sem1)
            scat(c0 + 1, rows1)
            return carry

        lax.fori_loop(0, (_NCH - 1) // 2, body, 0)
        wait(_NCH - 1, rows0, sem0)
        scat(_NCH - 1, rows0)

        plsc.subcore_barrier()
        pltpu.sync_copy(acc.at[pl.ds(sid * _RPS, _RPS)],
                        out.at[cid, pl.ds(sid * _RPS, _RPS)])

    return sc_agg


_sc_agg_w1 = _make_sc_agg(_W1)
_sc_agg_d = _make_sc_agg(_D)


def _make_sc_edge():
    mesh = plsc.VectorSubcoreMesh(core_axis_name="c", subcore_axis_name="s")

    @functools.partial(
        pl.kernel,
        out_type=jax.ShapeDtypeStruct((_E, _NCLS), jnp.float32),
        mesh=mesh,
        compiler_params=pltpu.CompilerParams(use_tc_tiling_on_sc=False),
        scratch_types=[
            pltpu.VMEM((_NCH, _CK), jnp.int32),
            pltpu.VMEM((_NCH, _CK), jnp.int32),
            pltpu.VMEM((_CK, _NCLS), jnp.float32),
            pltpu.VMEM((_CK, _NCLS), jnp.float32),
            pltpu.VMEM((_CK, _NCLS), jnp.float32),
            pltpu.VMEM((_CK, _NCLS), jnp.float32),
            pltpu.SemaphoreType.DMA,
            pltpu.SemaphoreType.DMA,
        ],
    )
    def sc_edge(pu, pv, src, dst, out, src_v, dst_v, u0, v0, u1, v1,
                sem0, sem1):
        cid = lax.axis_index("c")
        sid = lax.axis_index("s")
        wid = cid * _NS + sid
        base = wid * _EPT
        pltpu.sync_copy(src.at[wid], src_v)
        pltpu.sync_copy(dst.at[wid], dst_v)

        def gather2(c, u, v, sem):
            pltpu.async_copy(pu.at[src_v.at[c]], u, sem)
            pltpu.async_copy(pv.at[dst_v.at[c]], v, sem)

        def wait2(c, u, v, sem):
            pltpu.make_async_copy(pu.at[src_v.at[c]], u, sem).wait()
            pltpu.make_async_copy(pv.at[dst_v.at[c]], v, sem).wait()

        def compute_store(c, u, v):
            def add_row(r, carry):
                for j in range(_NCLS // 16):
                    sl = pl.ds(j * 16, 16)
                    u[r, sl] = u[r, sl] + v[r, sl]
                return carry

            lax.fori_loop(0, _CK, add_row, 0)
            pltpu.sync_copy(u, out.at[pl.ds(base + c * _CK, _CK)])

        gather2(0, u0, v0, sem0)

        def body(i, carry):
            c0 = 2 * i
            gather2(c0 + 1, u1, v1, sem1)
            wait2(c0, u0, v0, sem0)
            compute_store(c0, u0, v0)
            gather2(c0 + 2, u0, v0, sem0)
            wait2(c0 + 1, u1, v1, sem1)
            compute_store(c0 + 1, u1, v1)
            return carry

        lax.fori_loop(0, (_NCH - 1) // 2, body, 0)
        wait2(_NCH - 1, u0, v0, sem0)
        compute_store(_NCH - 1, u0, v0)

    return sc_edge


_sc_edge = _make_sc_edge()


def _tc1_body(x_ref, a_ref, ws_ref, wn_ref, b_ref, h1_ref, inv_ref):
    agg = a_ref[0] + a_ref[1]
    inv = 1.0 / jnp.clip(agg[:, _D:_D + 1], 1.0, None)
    neigh = agg[:, :_D] * inv
    h = jnp.dot(x_ref[...], ws_ref[...], preferred_element_type=jnp.float32)
    h = h + jnp.dot(neigh, wn_ref[...], preferred_element_type=jnp.float32)
    h = h + b_ref[...]
    h1_ref[...] = jnp.maximum(h, 0.0)
    inv_ref[...] = jnp.broadcast_to(inv, (inv.shape[0], _D))


def _tc1_call(x, aggp, ws1, wn1, b1):
    grid = (_N // _RB,)
    return pl.pallas_call(
        _tc1_body,
        grid=grid,
        in_specs=[
            pl.BlockSpec((_RB, _D), lambda i: (i, 0)),
            pl.BlockSpec((_NC, _RB, _W1), lambda i: (0, i, 0)),
            pl.BlockSpec((_D, _D), lambda i: (0, 0)),
            pl.BlockSpec((_D, _D), lambda i: (0, 0)),
            pl.BlockSpec((_D,), lambda i: (0,)),
        ],
        out_specs=[
            pl.BlockSpec((_RB, _D), lambda i: (i, 0)),
            pl.BlockSpec((_RB, _D), lambda i: (i, 0)),
        ],
        out_shape=[
            jax.ShapeDtypeStruct((_N, _D), jnp.float32),
            jax.ShapeDtypeStruct((_N, _D), jnp.float32),
        ],
    )(x, aggp, ws1, wn1, b1)


def _tc2_body(h1_ref, a_ref, inv_ref, ws_ref, wn_ref, b_ref, wpu_ref, wpv_ref,
              bp_ref, pu_ref, pv_ref):
    neigh = (a_ref[0] + a_ref[1]) * inv_ref[...]
    h2 = jnp.dot(h1_ref[...], ws_ref[...], preferred_element_type=jnp.float32)
    h2 = h2 + jnp.dot(neigh, wn_ref[...], preferred_element_type=jnp.float32)
    h2 = h2 + b_ref[...]
    pu_ref[...] = jnp.dot(h2, wpu_ref[...], preferred_element_type=jnp.float32)
    pv_ref[...] = jnp.dot(h2, wpv_ref[...],
                          preferred_element_type=jnp.float32) + bp_ref[...]


def _tc2_call(h1, aggp, invdeg, ws2, wn2, b2, wpu, wpv, bp):
    grid = (_N // _RB,)
    return pl.pallas_call(
        _tc2_body,
        grid=grid,
        in_specs=[
            pl.BlockSpec((_RB, _D), lambda i: (i, 0)),
            pl.BlockSpec((_NC, _RB, _D), lambda i: (0, i, 0)),
            pl.BlockSpec((_RB, _D), lambda i: (i, 0)),
            pl.BlockSpec((_D, _D), lambda i: (0, 0)),
            pl.BlockSpec((_D, _D), lambda i: (0, 0)),
            pl.BlockSpec((_D,), lambda i: (0,)),
            pl.BlockSpec((_D, _NCLS), lambda i: (0, 0)),
            pl.BlockSpec((_D, _NCLS), lambda i: (0, 0)),
            pl.BlockSpec((_NCLS,), lambda i: (0,)),
        ],
        out_specs=[
            pl.BlockSpec((_RB, _NCLS), lambda i: (i, 0)),
            pl.BlockSpec((_RB, _NCLS), lambda i: (i, 0)),
        ],
        out_shape=[
            jax.ShapeDtypeStruct((_N, _NCLS), jnp.float32),
            jax.ShapeDtypeStruct((_N, _NCLS), jnp.float32),
        ],
    )(h1, aggp, invdeg, ws2, wn2, b2, wpu, wpv, bp)


def kernel(x, edge_index, W_self1, W_neigh1, b1, W_self2, W_neigh2, b2,
           W_pred, b_pred):
    src = edge_index[0].reshape(_NW, _NCH, _CK)
    dst = edge_index[1].reshape(_NW, _NCH, _CK)

    xt = jnp.concatenate(
        [x, jnp.ones((_N, 1), jnp.float32),
         jnp.zeros((_N, _W1 - _D - 1), jnp.float32)], axis=1)

    aggp1 = _sc_agg_w1(xt, src, dst, jnp.zeros((_N, _W1), jnp.float32))
    h1, invdeg = _tc1_call(x, aggp1, W_self1, W_neigh1, b1)
    aggp2 = _sc_agg_d(h1, src, dst, jnp.zeros((_N, _D), jnp.float32))
    pu, pv = _tc2_call(h1, aggp2, invdeg, W_self2, W_neigh2, b2,
                       W_pred[:_D], W_pred[_D:], b_pred)
    return _sc_edge(pu, pv, src, dst)
